# Initial kernel scaffold; baseline (speedup 1.0000x reference)
import jax
import jax.numpy as jnp
from jax.experimental import pallas as pl


def kernel(t, mask, x2d, g1, g2, g3, adj1, adj2, adj3, d_adj, params):
    raise NotImplementedError("write your pallas kernel here")


# speedup vs baseline: 28.0042x; 28.0042x over previous
import functools

import jax
import jax.numpy as jnp
import numpy as np
from jax.experimental import pallas as pl
from jax.experimental.pallas import tpu as pltpu

_T = 32
_P = 3
_D = 256
_NG = 5
_J2 = 6
_J3 = 3
_K = 2
_DT = 0.025

_RT = 3
_TT = 18
_LT = 45

_S2_NP = np.tile(np.eye(_P, dtype=np.float32), (_J2, 1))
_S3P_NP = np.zeros((_LT, _TT), np.float32)
for _g in range(_NG):
    for _k in range(_J3):
        for _p in range(_P):
            _S3P_NP[_g * 9 + _k * 3 + _p, (_g + 1) * 3 + _p] = 1.0


def _dot(a, b):
    return jnp.dot(a, b, preferred_element_type=jnp.float32)


def _xp(x2, w2):
    return x2[:, 0:1] * w2[0:1, :] + x2[:, 1:2] * w2[1:2, :]


def _step(xr_ref, mr_ref, xt_ref, mt_ref, xl_ref, ml_ref,
          a1_ref, a2_ref, a3_ref, s2_ref, s3p_ref,
          w256_ref, w512_ref, xw512_ref, xw256_ref, b256_ref, b512_ref,
          wout_ref, bout_ref, h0_ref,
          yr_ref, yt_ref, yl_ref,
          hr_s, ht_s, hl_s):
    t = pl.program_id(0)

    @pl.when(t == 0)
    def _init():
        h0 = h0_ref[...]
        hr_s[...] = jnp.broadcast_to(h0, (_RT, _D))
        ht_s[...] = jnp.broadcast_to(h0, (_TT, _D))
        hl_s[...] = jnp.broadcast_to(h0, (_LT, _D))

    A1 = a1_ref[...]
    A2 = a2_ref[...]
    A3 = a3_ref[...]
    S2 = s2_ref[...]
    S3P = s3p_ref[...]

    W = lambda i: w256_ref[i]
    Z = lambda i: w512_ref[i]
    B = lambda i: b256_ref[i]
    BZ = lambda i: b512_ref[i]

    hr = hr_s[...]
    ht = ht_s[...]
    hl = hl_s[...]

    xr = xr_ref[0] * mr_ref[0]
    xt = xt_ref[0] * mt_ref[0]
    xl = xl_ref[0] * ml_ref[0]

    g = jnp.tanh(_dot(A1, _dot(hr, W(0))) + B(0))
    g = jnp.tanh(_dot(A1, _dot(g, W(1))) + B(1))
    dh = _dot(A1, _dot(g, W(2))) + B(2)
    hr1 = hr + dh * (2.0 * _K * _DT)

    par2 = _dot(S2, _dot(hr1, W(3)))
    for _ in range(_K):
        u = jnp.tanh(_dot(A2, par2 + _dot(ht, W(4))) + B(3))
        u = jnp.tanh(_dot(A2, _dot(u, W(5))) + B(4))
        d2 = _dot(A2, _dot(u, W(6))) + B(5)
        ht = ht + d2 * (_K * _DT)
        par3 = _dot(S3P, _dot(ht, W(7)))
        for _ in range(_K):
            v = jnp.tanh(_dot(A3, par3 + _dot(hl, W(8))) + B(6))
            v = jnp.tanh(_dot(A3, _dot(v, W(9))) + B(7))
            d3 = _dot(A3, _dot(v, W(10))) + B(8)
            hl = hl + d3 * _DT

    zr = jax.nn.sigmoid(_dot(A1, _xp(xr, xw512_ref[0]) + _dot(hr1, Z(0))) + BZ(0))
    z, r = zr[:, :_D], zr[:, _D:]
    hh = jnp.tanh(_dot(A1, _xp(xr, xw256_ref[0]) + _dot(r * hr1, W(11))) + B(9))
    h21 = z * hr1 + (1.0 - z) * hh
    zr = jax.nn.sigmoid(_dot(A1, _dot(h21, Z(1)) + _dot(hr1, Z(2))) + BZ(1))
    z, r = zr[:, :_D], zr[:, _D:]
    hh = jnp.tanh(_dot(A1, _dot(h21, W(12)) + _dot(r * hr1, W(13))) + B(10))
    h21 = z * hr1 + (1.0 - z) * hh

    zr = jax.nn.sigmoid(
        _dot(A2, _dot(S2, _dot(h21, Z(3))) + _xp(xt, xw512_ref[1]) + _dot(ht, Z(4))) + BZ(2))
    z, r = zr[:, :_D], zr[:, _D:]
    hh = jnp.tanh(
        _dot(A2, _dot(S2, _dot(h21, W(14))) + _xp(xt, xw256_ref[1]) + _dot(r * ht, W(15))) + B(11))
    h22 = z * ht + (1.0 - z) * hh
    zr = jax.nn.sigmoid(_dot(A2, _dot(h22, Z(5)) + _dot(ht, Z(6))) + BZ(3))
    z, r = zr[:, :_D], zr[:, _D:]
    hh = jnp.tanh(_dot(A2, _dot(h22, W(16)) + _dot(r * ht, W(17))) + B(12))
    h22 = z * ht + (1.0 - z) * hh

    zr = jax.nn.sigmoid(
        _dot(A3, _dot(S3P, _dot(h22, Z(7))) + _xp(xl, xw512_ref[2]) + _dot(hl, Z(8))) + BZ(4))
    z, r = zr[:, :_D], zr[:, _D:]
    hh = jnp.tanh(
        _dot(A3, _dot(S3P, _dot(h22, W(18))) + _xp(xl, xw256_ref[2]) + _dot(r * hl, W(19))) + B(13))
    h23 = z * hl + (1.0 - z) * hh
    zr = jax.nn.sigmoid(_dot(A3, _dot(h23, Z(9)) + _dot(hl, Z(10))) + BZ(5))
    z, r = zr[:, :_D], zr[:, _D:]
    hh = jnp.tanh(_dot(A3, _dot(h23, W(20)) + _dot(r * hl, W(21))) + B(14))
    h23 = z * hl + (1.0 - z) * hh

    hr_s[...] = h21
    ht_s[...] = h22
    hl_s[...] = h23

    wout = wout_ref[...]
    bout = bout_ref[...]
    yr_ref[0] = _dot(h21, wout) + bout
    yt_ref[0] = _dot(h22, wout) + bout
    yl_ref[0] = _dot(h23, wout) + bout


def kernel(t, mask, x2d, g1, g2, g3, adj1, adj2, adj3, d_adj, params):
    f32 = jnp.float32
    eP = jnp.eye(_P, dtype=f32)

    A1 = adj1[0]
    A2 = jnp.einsum("jk,pq->jpkq", adj2[0], eP).reshape(_TT, _TT)
    A3 = jnp.einsum("gkl,gh,pq->gkphlq", adj3[0],
                    jnp.eye(_NG, dtype=f32), eP).reshape(_LT, _LT)
    S2 = jnp.asarray(_S2_NP)
    S3P = jnp.asarray(_S3P_NP)

    xj = jnp.transpose(x2d[0], (0, 2, 1, 3))
    mj = jnp.transpose(mask[0], (0, 2, 1, 3))
    xr = xj[:, 0]
    xt = xj[:, 1:7].reshape(_T, _TT, 2)
    xl = xj[:, 7:22].reshape(_T, _LT, 2)
    mr = mj[:, 0]
    mt = mj[:, 1:7].reshape(_T, _TT, 1)
    ml = mj[:, 7:22].reshape(_T, _LT, 1)

    def gru_split(cell, segs):
        Wz, Wr, Wh = cell["Wz"], cell["Wr"], cell["Wh"]
        out_zr, out_h = [], []
        o = 0
        for s in segs:
            out_zr.append(jnp.concatenate([Wz[o:o + s], Wr[o:o + s]], axis=1))
            out_h.append(Wh[o:o + s])
            o += s
        bzr = jnp.concatenate([cell["bz"], cell["br"]]).reshape(1, 2 * _D)
        bh = cell["bh"].reshape(1, _D)
        return out_zr, out_h, bzr, bh

    p = params
    o1 = p["ODE1"]; o2 = p["ODE2"]; o3 = p["ODE3"]
    g1c0_zr, g1c0_h, g1c0_bzr, g1c0_bh = gru_split(p["GRU1"][0], [2, _D])
    g1c1_zr, g1c1_h, g1c1_bzr, g1c1_bh = gru_split(p["GRU1"][1], [_D, _D])
    g2c0_zr, g2c0_h, g2c0_bzr, g2c0_bh = gru_split(p["GRU2"][0], [_D, 2, _D])
    g2c1_zr, g2c1_h, g2c1_bzr, g2c1_bh = gru_split(p["GRU2"][1], [_D, _D])
    g3c0_zr, g3c0_h, g3c0_bzr, g3c0_bh = gru_split(p["GRU3"][0], [_D, 2, _D])
    g3c1_zr, g3c1_h, g3c1_bzr, g3c1_bh = gru_split(p["GRU3"][1], [_D, _D])

    W256 = jnp.stack([
        o1["W"][0], o1["W"][1], o1["W"][2],
        o2["W"][0][:_D], o2["W"][0][_D:], o2["W"][1], o2["W"][2],
        o3["W"][0][:_D], o3["W"][0][_D:], o3["W"][1], o3["W"][2],
        g1c0_h[1],
        g1c1_h[0], g1c1_h[1],
        g2c0_h[0], g2c0_h[2],
        g2c1_h[0], g2c1_h[1],
        g3c0_h[0], g3c0_h[2],
        g3c1_h[0], g3c1_h[1],
    ])
    W512 = jnp.stack([
        g1c0_zr[1],
        g1c1_zr[0], g1c1_zr[1],
        g2c0_zr[0], g2c0_zr[2],
        g2c1_zr[0], g2c1_zr[1],
        g3c0_zr[0], g3c0_zr[2],
        g3c1_zr[0], g3c1_zr[1],
    ])
    XW512 = jnp.stack([g1c0_zr[0], g2c0_zr[1], g3c0_zr[1]])
    XW256 = jnp.stack([g1c0_h[0], g2c0_h[1], g3c0_h[1]])
    B256 = jnp.stack([
        o1["b"][0].reshape(1, _D), o1["b"][1].reshape(1, _D), o1["b"][2].reshape(1, _D),
        o2["b"][0].reshape(1, _D), o2["b"][1].reshape(1, _D), o2["b"][2].reshape(1, _D),
        o3["b"][0].reshape(1, _D), o3["b"][1].reshape(1, _D), o3["b"][2].reshape(1, _D),
        g1c0_bh, g1c1_bh, g2c0_bh, g2c1_bh, g3c0_bh, g3c1_bh,
    ])
    B512 = jnp.stack([g1c0_bzr, g1c1_bzr, g2c0_bzr, g2c1_bzr, g3c0_bzr, g3c1_bzr])

    wout = p["Wout"]
    bout = p["bout"].reshape(1, 3)
    h0 = p["h0"].reshape(1, _D)

    def fixed(a):
        return pl.BlockSpec(a.shape, lambda i, _n=a.ndim: (0,) * _n)

    yr, yt, yl = pl.pallas_call(
        _step,
        grid=(_T,),
        in_specs=[
            pl.BlockSpec((1, _RT, 2), lambda i: (i, 0, 0)),
            pl.BlockSpec((1, _RT, 1), lambda i: (i, 0, 0)),
            pl.BlockSpec((1, _TT, 2), lambda i: (i, 0, 0)),
            pl.BlockSpec((1, _TT, 1), lambda i: (i, 0, 0)),
            pl.BlockSpec((1, _LT, 2), lambda i: (i, 0, 0)),
            pl.BlockSpec((1, _LT, 1), lambda i: (i, 0, 0)),
            fixed(A1), fixed(A2), fixed(A3), fixed(S2), fixed(S3P),
            fixed(W256), fixed(W512), fixed(XW512), fixed(XW256),
            fixed(B256), fixed(B512), fixed(wout), fixed(bout), fixed(h0),
        ],
        out_specs=[
            pl.BlockSpec((1, _RT, 3), lambda i: (i, 0, 0)),
            pl.BlockSpec((1, _TT, 3), lambda i: (i, 0, 0)),
            pl.BlockSpec((1, _LT, 3), lambda i: (i, 0, 0)),
        ],
        out_shape=[
            jax.ShapeDtypeStruct((_T, _RT, 3), f32),
            jax.ShapeDtypeStruct((_T, _TT, 3), f32),
            jax.ShapeDtypeStruct((_T, _LT, 3), f32),
        ],
        scratch_shapes=[
            pltpu.VMEM((_RT, _D), f32),
            pltpu.VMEM((_TT, _D), f32),
            pltpu.VMEM((_LT, _D), f32),
        ],
        compiler_params=pltpu.CompilerParams(
            dimension_semantics=("arbitrary",)),
    )(xr, mr, xt, mt, xl, ml, A1, A2, A3, S2, S3P,
      W256, W512, XW512, XW256, B256, B512, wout, bout, h0)

    y = jnp.concatenate([
        yr.reshape(_T, 1, _P, 3),
        yt.reshape(_T, _J2, _P, 3),
        yl.reshape(_T, _NG * _J3, _P, 3),
    ], axis=1)
    return jnp.transpose(y, (0, 2, 1, 3))[None]


# speedup vs baseline: 38.1939x; 1.3639x over previous
import functools

import jax
import jax.numpy as jnp
import numpy as np
from jax.experimental import pallas as pl
from jax.experimental.pallas import tpu as pltpu

_T = 32
_P = 3
_D = 256
_NG = 5
_J2 = 6
_J3 = 3
_K = 2
_DT = 0.025

_LR = _NG * _P


def _dot(a, b):
    return jnp.dot(a, b, preferred_element_type=jnp.float32)


def _xp(x2, w2):
    return x2[:, 0:1] * w2[0:1, :] + x2[:, 1:2] * w2[1:2, :]


def _step(xr_ref, mr_ref, xt_ref, mt_ref, xl_ref, ml_ref,
          a1_ref, a2_ref, c3_ref,
          w256_ref, w512_ref, xw512_ref, xw256_ref, b256_ref, b512_ref,
          wout_ref, bout_ref, h0_ref,
          yr_ref, yt_ref, yl_ref,
          hr_s, ht_s, hl_s):
    t = pl.program_id(0)

    @pl.when(t == 0)
    def _init():
        h0 = h0_ref[...]
        hr_s[...] = jnp.broadcast_to(h0, (_P, _D))
        ht_s[...] = jnp.broadcast_to(h0, (_J2, _P, _D))
        hl_s[...] = jnp.broadcast_to(h0, (_J3, _LR, _D))

    A1 = a1_ref[...]
    A2 = a2_ref[...]

    def mixA1(x):
        return (A1[:, 0:1] * x[0:1] + A1[:, 1:2] * x[1:2]
                + A1[:, 2:3] * x[2:3])

    def mixA2(ys):
        out = []
        for j in range(_J2):
            acc = A2[j:j + 1, 0:1] * ys[0]
            for q in range(1, _J2):
                acc = acc + A2[j:j + 1, q:q + 1] * ys[q]
            out.append(acc)
        return out

    def mixA3(ys):
        return [c3_ref[k, 0] * ys[0] + c3_ref[k, 1] * ys[1]
                + c3_ref[k, 2] * ys[2] for k in range(_J3)]

    W = lambda i: w256_ref[i]
    Z = lambda i: w512_ref[i]
    B = lambda i: b256_ref[i]
    BZ = lambda i: b512_ref[i]

    hr = hr_s[...]
    htj = [ht_s[j] for j in range(_J2)]
    hlk = [hl_s[k] for k in range(_J3)]

    xr = xr_ref[0] * mr_ref[0]
    xtv, mtv = xt_ref[0], mt_ref[0]
    xts = [xtv[j] * mtv[j] for j in range(_J2)]
    xlv, mlv = xl_ref[0], ml_ref[0]
    xls = [xlv[k] * mlv[k] for k in range(_J3)]

    g = jnp.tanh(mixA1(_dot(hr, W(0))) + B(0))
    g = jnp.tanh(mixA1(_dot(g, W(1))) + B(1))
    dh = mixA1(_dot(g, W(2))) + B(2)
    hr1 = hr + dh * (2.0 * _K * _DT)

    par2 = _dot(hr1, W(3))
    for _ in range(_K):
        y = mixA2([_dot(htj[j], W(4)) for j in range(_J2)])
        u = [jnp.tanh(par2 + y[j] + B(3)) for j in range(_J2)]
        y = mixA2([_dot(u[j], W(5)) for j in range(_J2)])
        u = [jnp.tanh(y[j] + B(4)) for j in range(_J2)]
        y = mixA2([_dot(u[j], W(6)) for j in range(_J2)])
        htj = [htj[j] + (y[j] + B(5)) * (_K * _DT) for j in range(_J2)]
        tcat = jnp.concatenate(htj[1:], axis=0)
        par3 = _dot(tcat, W(7))
        for _ in range(_K):
            y = mixA3([_dot(hlk[k], W(8)) for k in range(_J3)])
            v = [jnp.tanh(par3 + y[k] + B(6)) for k in range(_J3)]
            y = mixA3([_dot(v[k], W(9)) for k in range(_J3)])
            v = [jnp.tanh(y[k] + B(7)) for k in range(_J3)]
            y = mixA3([_dot(v[k], W(10)) for k in range(_J3)])
            hlk = [hlk[k] + (y[k] + B(8)) * _DT for k in range(_J3)]

    zr = jax.nn.sigmoid(mixA1(_xp(xr, xw512_ref[0]) + _dot(hr1, Z(0))) + BZ(0))
    z, r = zr[:, :_D], zr[:, _D:]
    hh = jnp.tanh(mixA1(_xp(xr, xw256_ref[0]) + _dot(r * hr1, W(11))) + B(9))
    h21 = z * hr1 + (1.0 - z) * hh
    zr = jax.nn.sigmoid(mixA1(_dot(h21, Z(1)) + _dot(hr1, Z(2))) + BZ(1))
    z, r = zr[:, :_D], zr[:, _D:]
    hh = jnp.tanh(mixA1(_dot(h21, W(12)) + _dot(r * hr1, W(13))) + B(10))
    h21 = z * hr1 + (1.0 - z) * hh

    parz = _dot(h21, Z(3))
    y = mixA2([_dot(htj[j], Z(4)) + _xp(xts[j], xw512_ref[1])
               for j in range(_J2)])
    zrs = [jax.nn.sigmoid(parz + y[j] + BZ(2)) for j in range(_J2)]
    parh = _dot(h21, W(14))
    y = mixA2([_dot(zrs[j][:, _D:] * htj[j], W(15)) + _xp(xts[j], xw256_ref[1])
               for j in range(_J2)])
    h22 = [zrs[j][:, :_D] * htj[j]
           + (1.0 - zrs[j][:, :_D]) * jnp.tanh(parh + y[j] + B(11))
           for j in range(_J2)]
    y = mixA2([_dot(h22[j], Z(5)) + _dot(htj[j], Z(6)) for j in range(_J2)])
    zrs = [jax.nn.sigmoid(y[j] + BZ(3)) for j in range(_J2)]
    y = mixA2([_dot(h22[j], W(16)) + _dot(zrs[j][:, _D:] * htj[j], W(17))
               for j in range(_J2)])
    h22 = [zrs[j][:, :_D] * htj[j]
           + (1.0 - zrs[j][:, :_D]) * jnp.tanh(y[j] + B(12))
           for j in range(_J2)]

    t22 = jnp.concatenate(h22[1:], axis=0)
    parz = _dot(t22, Z(7))
    y = mixA3([_dot(hlk[k], Z(8)) + _xp(xls[k], xw512_ref[2])
               for k in range(_J3)])
    zrs = [jax.nn.sigmoid(parz + y[k] + BZ(4)) for k in range(_J3)]
    parh = _dot(t22, W(18))
    y = mixA3([_dot(zrs[k][:, _D:] * hlk[k], W(19)) + _xp(xls[k], xw256_ref[2])
               for k in range(_J3)])
    h23 = [zrs[k][:, :_D] * hlk[k]
           + (1.0 - zrs[k][:, :_D]) * jnp.tanh(parh + y[k] + B(13))
           for k in range(_J3)]
    y = mixA3([_dot(h23[k], Z(9)) + _dot(hlk[k], Z(10)) for k in range(_J3)])
    zrs = [jax.nn.sigmoid(y[k] + BZ(5)) for k in range(_J3)]
    y = mixA3([_dot(h23[k], W(20)) + _dot(zrs[k][:, _D:] * hlk[k], W(21))
               for k in range(_J3)])
    h23 = [zrs[k][:, :_D] * hlk[k]
           + (1.0 - zrs[k][:, :_D]) * jnp.tanh(y[k] + B(14))
           for k in range(_J3)]

    hr_s[...] = h21
    for j in range(_J2):
        ht_s[j] = h22[j]
    for k in range(_J3):
        hl_s[k] = h23[k]

    wout = wout_ref[...]
    bout = bout_ref[...]
    yr_ref[0] = _dot(h21, wout) + bout
    yt_ref[0] = _dot(jnp.concatenate(h22, axis=0), wout) + bout
    for k in range(_J3):
        yl_ref[0, k] = _dot(h23[k], wout) + bout


def kernel(t, mask, x2d, g1, g2, g3, adj1, adj2, adj3, d_adj, params):
    f32 = jnp.float32

    A1 = adj1[0]
    A2 = adj2[0]
    C3 = jnp.reshape(
        jnp.broadcast_to(jnp.transpose(adj3[0], (1, 2, 0))[:, :, :, None, None],
                         (_J3, _J3, _NG, _P, 1)),
        (_J3, _J3, _LR, 1))

    xj = jnp.transpose(x2d[0], (0, 2, 1, 3))
    mj = jnp.transpose(mask[0], (0, 2, 1, 3))
    xr = xj[:, 0]
    xt = xj[:, 1:7]
    xl = jnp.transpose(xj[:, 7:22].reshape(_T, _NG, _J3, _P, 2),
                       (0, 2, 1, 3, 4)).reshape(_T, _J3, _LR, 2)
    mr = mj[:, 0]
    mt = mj[:, 1:7]
    ml = jnp.transpose(mj[:, 7:22].reshape(_T, _NG, _J3, _P, 1),
                       (0, 2, 1, 3, 4)).reshape(_T, _J3, _LR, 1)

    def gru_split(cell, segs):
        Wz, Wr, Wh = cell["Wz"], cell["Wr"], cell["Wh"]
        out_zr, out_h = [], []
        o = 0
        for s in segs:
            out_zr.append(jnp.concatenate([Wz[o:o + s], Wr[o:o + s]], axis=1))
            out_h.append(Wh[o:o + s])
            o += s
        bzr = jnp.concatenate([cell["bz"], cell["br"]]).reshape(1, 2 * _D)
        bh = cell["bh"].reshape(1, _D)
        return out_zr, out_h, bzr, bh

    p = params
    o1 = p["ODE1"]; o2 = p["ODE2"]; o3 = p["ODE3"]
    g1c0_zr, g1c0_h, g1c0_bzr, g1c0_bh = gru_split(p["GRU1"][0], [2, _D])
    g1c1_zr, g1c1_h, g1c1_bzr, g1c1_bh = gru_split(p["GRU1"][1], [_D, _D])
    g2c0_zr, g2c0_h, g2c0_bzr, g2c0_bh = gru_split(p["GRU2"][0], [_D, 2, _D])
    g2c1_zr, g2c1_h, g2c1_bzr, g2c1_bh = gru_split(p["GRU2"][1], [_D, _D])
    g3c0_zr, g3c0_h, g3c0_bzr, g3c0_bh = gru_split(p["GRU3"][0], [_D, 2, _D])
    g3c1_zr, g3c1_h, g3c1_bzr, g3c1_bh = gru_split(p["GRU3"][1], [_D, _D])

    W256 = jnp.stack([
        o1["W"][0], o1["W"][1], o1["W"][2],
        o2["W"][0][:_D], o2["W"][0][_D:], o2["W"][1], o2["W"][2],
        o3["W"][0][:_D], o3["W"][0][_D:], o3["W"][1], o3["W"][2],
        g1c0_h[1],
        g1c1_h[0], g1c1_h[1],
        g2c0_h[0], g2c0_h[2],
        g2c1_h[0], g2c1_h[1],
        g3c0_h[0], g3c0_h[2],
        g3c1_h[0], g3c1_h[1],
    ])
    W512 = jnp.stack([
        g1c0_zr[1],
        g1c1_zr[0], g1c1_zr[1],
        g2c0_zr[0], g2c0_zr[2],
        g2c1_zr[0], g2c1_zr[1],
        g3c0_zr[0], g3c0_zr[2],
        g3c1_zr[0], g3c1_zr[1],
    ])
    XW512 = jnp.stack([g1c0_zr[0], g2c0_zr[1], g3c0_zr[1]])
    XW256 = jnp.stack([g1c0_h[0], g2c0_h[1], g3c0_h[1]])
    B256 = jnp.stack([
        o1["b"][0].reshape(1, _D), o1["b"][1].reshape(1, _D), o1["b"][2].reshape(1, _D),
        o2["b"][0].reshape(1, _D), o2["b"][1].reshape(1, _D), o2["b"][2].reshape(1, _D),
        o3["b"][0].reshape(1, _D), o3["b"][1].reshape(1, _D), o3["b"][2].reshape(1, _D),
        g1c0_bh, g1c1_bh, g2c0_bh, g2c1_bh, g3c0_bh, g3c1_bh,
    ])
    B512 = jnp.stack([g1c0_bzr, g1c1_bzr, g2c0_bzr, g2c1_bzr, g3c0_bzr, g3c1_bzr])

    wout = p["Wout"]
    bout = p["bout"].reshape(1, 3)
    h0 = p["h0"].reshape(1, _D)

    def fixed(a):
        return pl.BlockSpec(a.shape, lambda i, _n=a.ndim: (0,) * _n)

    yr, yt_o, yl_o = pl.pallas_call(
        _step,
        grid=(_T,),
        in_specs=[
            pl.BlockSpec((1, _P, 2), lambda i: (i, 0, 0)),
            pl.BlockSpec((1, _P, 1), lambda i: (i, 0, 0)),
            pl.BlockSpec((1, _J2, _P, 2), lambda i: (i, 0, 0, 0)),
            pl.BlockSpec((1, _J2, _P, 1), lambda i: (i, 0, 0, 0)),
            pl.BlockSpec((1, _J3, _LR, 2), lambda i: (i, 0, 0, 0)),
            pl.BlockSpec((1, _J3, _LR, 1), lambda i: (i, 0, 0, 0)),
            fixed(A1), fixed(A2), fixed(C3),
            fixed(W256), fixed(W512), fixed(XW512), fixed(XW256),
            fixed(B256), fixed(B512), fixed(wout), fixed(bout), fixed(h0),
        ],
        out_specs=[
            pl.BlockSpec((1, _P, 3), lambda i: (i, 0, 0)),
            pl.BlockSpec((1, _J2 * _P, 3), lambda i: (i, 0, 0)),
            pl.BlockSpec((1, _J3, _LR, 3), lambda i: (i, 0, 0, 0)),
        ],
        out_shape=[
            jax.ShapeDtypeStruct((_T, _P, 3), f32),
            jax.ShapeDtypeStruct((_T, _J2 * _P, 3), f32),
            jax.ShapeDtypeStruct((_T, _J3, _LR, 3), f32),
        ],
        scratch_shapes=[
            pltpu.VMEM((_P, _D), f32),
            pltpu.VMEM((_J2, _P, _D), f32),
            pltpu.VMEM((_J3, _LR, _D), f32),
        ],
        compiler_params=pltpu.CompilerParams(
            dimension_semantics=("arbitrary",)),
    )(xr, mr, xt, mt, xl, ml, A1, A2, C3,
      W256, W512, XW512, XW256, B256, B512, wout, bout, h0)

    yl = jnp.transpose(yl_o.reshape(_T, _J3, _NG, _P, 3),
                       (0, 2, 1, 3, 4)).reshape(_T, _NG * _J3 * _P, 3)
    y = jnp.concatenate([
        yr.reshape(_T, 1, _P, 3),
        yt_o.reshape(_T, _J2, _P, 3),
        yl.reshape(_T, _NG * _J3, _P, 3),
    ], axis=1)
    return jnp.transpose(y, (0, 2, 1, 3))[None]


# speedup vs baseline: 38.5124x; 1.0083x over previous
import functools

import jax
import jax.numpy as jnp
import numpy as np
from jax.experimental import pallas as pl
from jax.experimental.pallas import tpu as pltpu

_T = 32
_P = 3
_D = 256
_NG = 5
_J2 = 6
_J3 = 3
_K = 2
_DT = 0.025

_LR = _NG * _P


def _dot(a, b):
    return jnp.dot(a, b, preferred_element_type=jnp.float32)


def _xp(x2, w2):
    return x2[:, 0:1] * w2[0:1, :] + x2[:, 1:2] * w2[1:2, :]


def _seq(xr_ref, mr_ref, xt_ref, mt_ref, xl_ref, ml_ref,
         a1_ref, a2_ref, c3_ref,
         w256_ref, w512_ref, xw512_ref, xw256_ref, b256_ref, b512_ref,
         wout_ref, bout_ref, h0_ref,
         yr_ref, yt_ref, yl_ref):
    A1 = a1_ref[...]
    A2 = a2_ref[...]

    def mixA1(x):
        return (A1[:, 0:1] * x[0:1] + A1[:, 1:2] * x[1:2]
                + A1[:, 2:3] * x[2:3])

    def mixA2(ys):
        out = []
        for j in range(_J2):
            acc = A2[j:j + 1, 0:1] * ys[0]
            for q in range(1, _J2):
                acc = acc + A2[j:j + 1, q:q + 1] * ys[q]
            out.append(acc)
        return out

    def mixA3(ys):
        return [c3_ref[k, 0] * ys[0] + c3_ref[k, 1] * ys[1]
                + c3_ref[k, 2] * ys[2] for k in range(_J3)]

    W = lambda i: w256_ref[i]
    Z = lambda i: w512_ref[i]
    B = lambda i: b256_ref[i]
    BZ = lambda i: b512_ref[i]

    wout = wout_ref[...]
    bout = bout_ref[...]

    def body(t, carry):
        hr = carry[0]
        htj = list(carry[1:1 + _J2])
        hlk = list(carry[1 + _J2:])

        xr = xr_ref[t] * mr_ref[t]
        xtv, mtv = xt_ref[t], mt_ref[t]
        xts = [xtv[j] * mtv[j] for j in range(_J2)]
        xlv, mlv = xl_ref[t], ml_ref[t]
        xls = [xlv[k] * mlv[k] for k in range(_J3)]

        g = jnp.tanh(mixA1(_dot(hr, W(0))) + B(0))
        g = jnp.tanh(mixA1(_dot(g, W(1))) + B(1))
        dh = mixA1(_dot(g, W(2))) + B(2)
        hr1 = hr + dh * (2.0 * _K * _DT)

        par2 = _dot(hr1, W(3))
        for _ in range(_K):
            y = mixA2([_dot(htj[j], W(4)) for j in range(_J2)])
            u = [jnp.tanh(par2 + y[j] + B(3)) for j in range(_J2)]
            y = mixA2([_dot(u[j], W(5)) for j in range(_J2)])
            u = [jnp.tanh(y[j] + B(4)) for j in range(_J2)]
            y = mixA2([_dot(u[j], W(6)) for j in range(_J2)])
            htj = [htj[j] + (y[j] + B(5)) * (_K * _DT) for j in range(_J2)]
            tcat = jnp.concatenate(htj[1:], axis=0)
            par3 = _dot(tcat, W(7))
            for _ in range(_K):
                y = mixA3([_dot(hlk[k], W(8)) for k in range(_J3)])
                v = [jnp.tanh(par3 + y[k] + B(6)) for k in range(_J3)]
                y = mixA3([_dot(v[k], W(9)) for k in range(_J3)])
                v = [jnp.tanh(y[k] + B(7)) for k in range(_J3)]
                y = mixA3([_dot(v[k], W(10)) for k in range(_J3)])
                hlk = [hlk[k] + (y[k] + B(8)) * _DT for k in range(_J3)]

        zr = jax.nn.sigmoid(mixA1(_xp(xr, xw512_ref[0]) + _dot(hr1, Z(0))) + BZ(0))
        z, r = zr[:, :_D], zr[:, _D:]
        hh = jnp.tanh(mixA1(_xp(xr, xw256_ref[0]) + _dot(r * hr1, W(11))) + B(9))
        h21 = z * hr1 + (1.0 - z) * hh
        zr = jax.nn.sigmoid(mixA1(_dot(h21, Z(1)) + _dot(hr1, Z(2))) + BZ(1))
        z, r = zr[:, :_D], zr[:, _D:]
        hh = jnp.tanh(mixA1(_dot(h21, W(12)) + _dot(r * hr1, W(13))) + B(10))
        h21 = z * hr1 + (1.0 - z) * hh

        parz = _dot(h21, Z(3))
        y = mixA2([_dot(htj[j], Z(4)) + _xp(xts[j], xw512_ref[1])
                   for j in range(_J2)])
        zrs = [jax.nn.sigmoid(parz + y[j] + BZ(2)) for j in range(_J2)]
        parh = _dot(h21, W(14))
        y = mixA2([_dot(zrs[j][:, _D:] * htj[j], W(15)) + _xp(xts[j], xw256_ref[1])
                   for j in range(_J2)])
        h22 = [zrs[j][:, :_D] * htj[j]
               + (1.0 - zrs[j][:, :_D]) * jnp.tanh(parh + y[j] + B(11))
               for j in range(_J2)]
        y = mixA2([_dot(h22[j], Z(5)) + _dot(htj[j], Z(6)) for j in range(_J2)])
        zrs = [jax.nn.sigmoid(y[j] + BZ(3)) for j in range(_J2)]
        y = mixA2([_dot(h22[j], W(16)) + _dot(zrs[j][:, _D:] * htj[j], W(17))
                   for j in range(_J2)])
        h22 = [zrs[j][:, :_D] * htj[j]
               + (1.0 - zrs[j][:, :_D]) * jnp.tanh(y[j] + B(12))
               for j in range(_J2)]

        t22 = jnp.concatenate(h22[1:], axis=0)
        parz = _dot(t22, Z(7))
        y = mixA3([_dot(hlk[k], Z(8)) + _xp(xls[k], xw512_ref[2])
                   for k in range(_J3)])
        zrs = [jax.nn.sigmoid(parz + y[k] + BZ(4)) for k in range(_J3)]
        parh = _dot(t22, W(18))
        y = mixA3([_dot(zrs[k][:, _D:] * hlk[k], W(19)) + _xp(xls[k], xw256_ref[2])
                   for k in range(_J3)])
        h23 = [zrs[k][:, :_D] * hlk[k]
               + (1.0 - zrs[k][:, :_D]) * jnp.tanh(parh + y[k] + B(13))
               for k in range(_J3)]
        y = mixA3([_dot(h23[k], Z(9)) + _dot(hlk[k], Z(10)) for k in range(_J3)])
        zrs = [jax.nn.sigmoid(y[k] + BZ(5)) for k in range(_J3)]
        y = mixA3([_dot(h23[k], W(20)) + _dot(zrs[k][:, _D:] * hlk[k], W(21))
                   for k in range(_J3)])
        h23 = [zrs[k][:, :_D] * hlk[k]
               + (1.0 - zrs[k][:, :_D]) * jnp.tanh(y[k] + B(14))
               for k in range(_J3)]

        yr_ref[t] = _dot(h21, wout) + bout
        yt_ref[t] = _dot(jnp.concatenate(h22, axis=0), wout) + bout
        for k in range(_J3):
            yl_ref[t, k] = _dot(h23[k], wout) + bout
        return (h21, *h22, *h23)

    h0 = h0_ref[...]
    init = (jnp.broadcast_to(h0, (_P, _D)),
            *[jnp.broadcast_to(h0, (_P, _D)) for _ in range(_J2)],
            *[jnp.broadcast_to(h0, (_LR, _D)) for _ in range(_J3)])
    jax.lax.fori_loop(0, _T, body, init)


def kernel(t, mask, x2d, g1, g2, g3, adj1, adj2, adj3, d_adj, params):
    f32 = jnp.float32

    A1 = adj1[0]
    A2 = adj2[0]
    C3 = jnp.reshape(
        jnp.broadcast_to(jnp.transpose(adj3[0], (1, 2, 0))[:, :, :, None, None],
                         (_J3, _J3, _NG, _P, 1)),
        (_J3, _J3, _LR, 1))

    xj = jnp.transpose(x2d[0], (0, 2, 1, 3))
    mj = jnp.transpose(mask[0], (0, 2, 1, 3))
    xr = xj[:, 0]
    xt = xj[:, 1:7]
    xl = jnp.transpose(xj[:, 7:22].reshape(_T, _NG, _J3, _P, 2),
                       (0, 2, 1, 3, 4)).reshape(_T, _J3, _LR, 2)
    mr = mj[:, 0]
    mt = mj[:, 1:7]
    ml = jnp.transpose(mj[:, 7:22].reshape(_T, _NG, _J3, _P, 1),
                       (0, 2, 1, 3, 4)).reshape(_T, _J3, _LR, 1)

    def gru_split(cell, segs):
        Wz, Wr, Wh = cell["Wz"], cell["Wr"], cell["Wh"]
        out_zr, out_h = [], []
        o = 0
        for s in segs:
            out_zr.append(jnp.concatenate([Wz[o:o + s], Wr[o:o + s]], axis=1))
            out_h.append(Wh[o:o + s])
            o += s
        bzr = jnp.concatenate([cell["bz"], cell["br"]]).reshape(1, 2 * _D)
        bh = cell["bh"].reshape(1, _D)
        return out_zr, out_h, bzr, bh

    p = params
    o1 = p["ODE1"]; o2 = p["ODE2"]; o3 = p["ODE3"]
    g1c0_zr, g1c0_h, g1c0_bzr, g1c0_bh = gru_split(p["GRU1"][0], [2, _D])
    g1c1_zr, g1c1_h, g1c1_bzr, g1c1_bh = gru_split(p["GRU1"][1], [_D, _D])
    g2c0_zr, g2c0_h, g2c0_bzr, g2c0_bh = gru_split(p["GRU2"][0], [_D, 2, _D])
    g2c1_zr, g2c1_h, g2c1_bzr, g2c1_bh = gru_split(p["GRU2"][1], [_D, _D])
    g3c0_zr, g3c0_h, g3c0_bzr, g3c0_bh = gru_split(p["GRU3"][0], [_D, 2, _D])
    g3c1_zr, g3c1_h, g3c1_bzr, g3c1_bh = gru_split(p["GRU3"][1], [_D, _D])

    W256 = jnp.stack([
        o1["W"][0], o1["W"][1], o1["W"][2],
        o2["W"][0][:_D], o2["W"][0][_D:], o2["W"][1], o2["W"][2],
        o3["W"][0][:_D], o3["W"][0][_D:], o3["W"][1], o3["W"][2],
        g1c0_h[1],
        g1c1_h[0], g1c1_h[1],
        g2c0_h[0], g2c0_h[2],
        g2c1_h[0], g2c1_h[1],
        g3c0_h[0], g3c0_h[2],
        g3c1_h[0], g3c1_h[1],
    ])
    W512 = jnp.stack([
        g1c0_zr[1],
        g1c1_zr[0], g1c1_zr[1],
        g2c0_zr[0], g2c0_zr[2],
        g2c1_zr[0], g2c1_zr[1],
        g3c0_zr[0], g3c0_zr[2],
        g3c1_zr[0], g3c1_zr[1],
    ])
    XW512 = jnp.stack([g1c0_zr[0], g2c0_zr[1], g3c0_zr[1]])
    XW256 = jnp.stack([g1c0_h[0], g2c0_h[1], g3c0_h[1]])
    B256 = jnp.stack([
        o1["b"][0].reshape(1, _D), o1["b"][1].reshape(1, _D), o1["b"][2].reshape(1, _D),
        o2["b"][0].reshape(1, _D), o2["b"][1].reshape(1, _D), o2["b"][2].reshape(1, _D),
        o3["b"][0].reshape(1, _D), o3["b"][1].reshape(1, _D), o3["b"][2].reshape(1, _D),
        g1c0_bh, g1c1_bh, g2c0_bh, g2c1_bh, g3c0_bh, g3c1_bh,
    ])
    B512 = jnp.stack([g1c0_bzr, g1c1_bzr, g2c0_bzr, g2c1_bzr, g3c0_bzr, g3c1_bzr])

    wout = p["Wout"]
    bout = p["bout"].reshape(1, 3)
    h0 = p["h0"].reshape(1, _D)

    yr, yt_o, yl_o = pl.pallas_call(
        _seq,
        out_shape=[
            jax.ShapeDtypeStruct((_T, _P, 3), f32),
            jax.ShapeDtypeStruct((_T, _J2 * _P, 3), f32),
            jax.ShapeDtypeStruct((_T, _J3, _LR, 3), f32),
        ],
    )(xr, mr, xt, mt, xl, ml, A1, A2, C3,
      W256, W512, XW512, XW256, B256, B512, wout, bout, h0)

    yl = jnp.transpose(yl_o.reshape(_T, _J3, _NG, _P, 3),
                       (0, 2, 1, 3, 4)).reshape(_T, _NG * _J3 * _P, 3)
    y = jnp.concatenate([
        yr.reshape(_T, 1, _P, 3),
        yt_o.reshape(_T, _J2, _P, 3),
        yl.reshape(_T, _NG * _J3, _P, 3),
    ], axis=1)
    return jnp.transpose(y, (0, 2, 1, 3))[None]
